# baseline (device time: 479507 ns/iter reference)
import jax
import jax.numpy as jnp
from jax import lax
from jax.experimental import pallas as pl
from jax.experimental.pallas import tpu as pltpu

N_DEV = 4
SQ = 2048
SKV_LOC = 2048
HL = 8
DH = 128
DLOC = HL * DH
NPH = 4
BLK = 64
NG_Q = SQ // (NPH * BLK)
NG_KV = SKV_LOC // (NPH * BLK)
QROWS = NG_Q * BLK
KROWS = N_DEV * NG_KV * BLK
SCALE = 0.08838834764831843


def _fused_body(x_ref, wq_ref, wo_ref, k_ref, ks_ref, v_ref, vs_ref, out_ref,
                kgbuf, vgbuf, ksbuf, vsbuf, send_sems, recv_sems, copy_sems):
    my = lax.axis_index("i")

    barrier = pltpu.get_barrier_semaphore()
    for di in range(1, N_DEV):
        pl.semaphore_signal(
            barrier, inc=1,
            device_id=((my + di) % N_DEV,),
            device_id_type=pl.DeviceIdType.MESH,
        )
    pl.semaphore_wait(barrier, N_DEV - 1)

    pairs = ((k_ref, kgbuf), (v_ref, vgbuf))
    spairs = ((ks_ref, ksbuf), (vs_ref, vsbuf))
    SS = (N_DEV - 1) * 2 * NPH
    RS_ = N_DEV * 2 * NPH

    copies = {}
    sends = []
    recvs = {}
    srecvs = {}
    for t, (src, dst) in enumerate(spairs):
        c = pltpu.make_async_copy(
            src.at[my],
            dst.at[my],
            copy_sems.at[2 * NPH + t],
        )
        c.start()
        copies[("s", t)] = c
        for di in range(1, N_DEV):
            peer = (my + di) % N_DEV
            r = pltpu.make_async_remote_copy(
                src_ref=src.at[peer],
                dst_ref=dst.at[my],
                send_sem=send_sems.at[SS + (di - 1) * 2 + t],
                recv_sem=recv_sems.at[RS_ + my * 2 + t],
                device_id=(peer,),
                device_id_type=pl.DeviceIdType.MESH,
            )
            r.start()
            sends.append(r)
            rr = pltpu.make_async_remote_copy(
                src_ref=src.at[peer],
                dst_ref=dst.at[peer],
                send_sem=send_sems.at[SS + t],
                recv_sem=recv_sems.at[RS_ + peer * 2 + t],
                device_id=(peer,),
                device_id_type=pl.DeviceIdType.MESH,
            )
            srecvs[(di, t)] = rr

    for p in range(NPH):
        for t, (src, dst) in enumerate(pairs):
            c = pltpu.make_async_copy(
                src.at[:, p, :, pl.ds(my * DLOC, DLOC)],
                dst.at[p, my],
                copy_sems.at[t * NPH + p],
            )
            c.start()
            copies[(t, p)] = c
        for di in range(1, N_DEV):
            peer = (my + di) % N_DEV
            for t, (src, dst) in enumerate(pairs):
                r = pltpu.make_async_remote_copy(
                    src_ref=src.at[:, p, :, pl.ds(peer * DLOC, DLOC)],
                    dst_ref=dst.at[p, my],
                    send_sem=send_sems.at[((di - 1) * 2 + t) * NPH + p],
                    recv_sem=recv_sems.at[(my * 2 + t) * NPH + p],
                    device_id=(peer,),
                    device_id_type=pl.DeviceIdType.MESH,
                )
                r.start()
                sends.append(r)
                src_dev = (my + di) % N_DEV
                rr = pltpu.make_async_remote_copy(
                    src_ref=src.at[:, p, :, pl.ds(src_dev * DLOC, DLOC)],
                    dst_ref=dst.at[p, src_dev],
                    send_sem=send_sems.at[t * NPH + p],
                    recv_sem=recv_sems.at[(src_dev * 2 + t) * NPH + p],
                    device_id=(src_dev,),
                    device_id_type=pl.DeviceIdType.MESH,
                )
                recvs[(di, t, p)] = rr

    q = jnp.dot(x_ref[:], wq_ref[:], preferred_element_type=jnp.float32)
    q = (q * SCALE).astype(jnp.bfloat16)

    for t in range(2):
        copies[("s", t)].wait()
        for di in range(1, N_DEV):
            srecvs[(di, t)].wait_recv()
    ksall = ksbuf[...]
    vsall = vsbuf[...]

    ctx_p = [None] * NPH
    for p in range(NPH):
        for t in range(2):
            copies[(t, p)].wait()
            for di in range(1, N_DEV):
                recvs[(di, t, p)].wait_recv()
        ks = ksall[:, :, p, :, :].reshape(KROWS, HL)
        vs = vsall[:, :, p, :, :].reshape(KROWS, HL)
        kp = kgbuf[p].reshape(KROWS, DLOC).astype(jnp.bfloat16)
        vp = vgbuf[p].reshape(KROWS, DLOC).astype(jnp.bfloat16)

        qp = jnp.concatenate(
            [q[g * NPH * BLK + p * BLK: g * NPH * BLK + (p + 1) * BLK, :]
             for g in range(NG_Q)],
            axis=0,
        )

        ctx_h = []
        for h in range(HL):
            qh = qp[:, h * DH:(h + 1) * DH]
            kh = kp[:, h * DH:(h + 1) * DH] * ks[:, h:h + 1]
            vh = vp[:, h * DH:(h + 1) * DH] * vs[:, h:h + 1]
            s = lax.dot_general(
                qh, kh, (((1,), (1,)), ((), ())),
                preferred_element_type=jnp.float32,
            )
            m = jnp.max(s, axis=1, keepdims=True)
            e = jnp.exp(s - m)
            l = jnp.sum(e, axis=1, keepdims=True)
            w = (e / l).astype(jnp.bfloat16)
            ctx = jnp.dot(w, vh, preferred_element_type=jnp.float32)
            ctx_h.append(ctx.astype(jnp.bfloat16))
        ctx_p[p] = jnp.concatenate(ctx_h, axis=1)

    rows = []
    for g in range(NG_Q):
        for p in range(NPH):
            rows.append(ctx_p[p][g * BLK:(g + 1) * BLK, :])
    ctx = jnp.concatenate(rows, axis=0)

    out = jnp.dot(ctx, wo_ref[:], preferred_element_type=jnp.float32)
    out_ref[:] = out.astype(jnp.bfloat16)

    for r in sends:
        r.wait_send()


def _fused_attn(x2, wq, wo, k8, ksc, v8, vsc):
    gshape = (NPH, N_DEV, NG_KV, BLK, DLOC)
    sshape = (N_DEV, NG_KV, NPH, BLK, HL)
    return pl.pallas_call(
        _fused_body,
        out_shape=jax.ShapeDtypeStruct((SQ, 1024), jnp.bfloat16),
        in_specs=[
            pl.BlockSpec(memory_space=pltpu.VMEM),
            pl.BlockSpec(memory_space=pltpu.VMEM),
            pl.BlockSpec(memory_space=pltpu.VMEM),
            pl.BlockSpec(memory_space=pl.ANY),
            pl.BlockSpec(memory_space=pl.ANY),
            pl.BlockSpec(memory_space=pl.ANY),
            pl.BlockSpec(memory_space=pl.ANY),
        ],
        out_specs=pl.BlockSpec(memory_space=pltpu.VMEM),
        scratch_shapes=[
            pltpu.VMEM(gshape, jnp.int8),
            pltpu.VMEM(gshape, jnp.int8),
            pltpu.VMEM(sshape, jnp.bfloat16),
            pltpu.VMEM(sshape, jnp.bfloat16),
            pltpu.SemaphoreType.DMA(((N_DEV - 1) * 2 * NPH
                                     + (N_DEV - 1) * 2,)),
            pltpu.SemaphoreType.DMA((N_DEV * 2 * NPH
                                     + N_DEV * 2,)),
            pltpu.SemaphoreType.DMA((2 * NPH + 2,)),
        ],
        compiler_params=pltpu.CompilerParams(
            collective_id=0, vmem_limit_bytes=64 * 1024 * 1024),
    )(x2, wq, wo, k8, ksc, v8, vsc)


QR = SQ // N_DEV


def _ar_body(part_ref, out_ref, rsbuf, rs_send, rs_recv,
             ag_send, ag_recv, csem):
    my = lax.axis_index("i")

    barrier = pltpu.get_barrier_semaphore()
    for di in range(1, N_DEV):
        pl.semaphore_signal(
            barrier, inc=1,
            device_id=((my + di) % N_DEV,),
            device_id_type=pl.DeviceIdType.MESH,
        )
    pl.semaphore_wait(barrier, N_DEV - 1)

    c = pltpu.make_async_copy(
        part_ref.at[pl.ds(my * QR, QR)], rsbuf.at[my], csem)
    c.start()
    rs_sends = []
    for di in range(1, N_DEV):
        peer = (my + di) % N_DEV
        r = pltpu.make_async_remote_copy(
            src_ref=part_ref.at[pl.ds(peer * QR, QR)],
            dst_ref=rsbuf.at[my],
            send_sem=rs_send.at[di - 1],
            recv_sem=rs_recv.at[my],
            device_id=(peer,),
            device_id_type=pl.DeviceIdType.MESH,
        )
        r.start()
        rs_sends.append(r)

    c.wait()
    for di in range(1, N_DEV):
        src_dev = (my + di) % N_DEV
        r = pltpu.make_async_remote_copy(
            src_ref=part_ref.at[pl.ds(0, QR)],
            dst_ref=rsbuf.at[src_dev],
            send_sem=rs_send.at[0],
            recv_sem=rs_recv.at[src_dev],
            device_id=(src_dev,),
            device_id_type=pl.DeviceIdType.MESH,
        )
        r.wait_recv()
    for r in rs_sends:
        r.wait_send()

    acc = rsbuf[...]
    red = (acc[0].astype(jnp.float32) + acc[1].astype(jnp.float32)
           + acc[2].astype(jnp.float32) + acc[3].astype(jnp.float32))
    out_ref[pl.ds(my * QR, QR), :] = red.astype(jnp.bfloat16)

    ag_sends = []
    for di in range(1, N_DEV):
        peer = (my + di) % N_DEV
        r = pltpu.make_async_remote_copy(
            src_ref=out_ref.at[pl.ds(my * QR, QR)],
            dst_ref=out_ref.at[pl.ds(my * QR, QR)],
            send_sem=ag_send.at[di - 1],
            recv_sem=ag_recv.at[my],
            device_id=(peer,),
            device_id_type=pl.DeviceIdType.MESH,
        )
        r.start()
        ag_sends.append(r)
    for di in range(1, N_DEV):
        src_dev = (my + di) % N_DEV
        r = pltpu.make_async_remote_copy(
            src_ref=out_ref.at[pl.ds(src_dev * QR, QR)],
            dst_ref=out_ref.at[pl.ds(src_dev * QR, QR)],
            send_sem=ag_send.at[0],
            recv_sem=ag_recv.at[src_dev],
            device_id=(src_dev,),
            device_id_type=pl.DeviceIdType.MESH,
        )
        r.wait_recv()
    for r in ag_sends:
        r.wait_send()


def _allreduce(part):
    return pl.pallas_call(
        _ar_body,
        out_shape=jax.ShapeDtypeStruct((SQ, 1024), jnp.bfloat16),
        in_specs=[pl.BlockSpec(memory_space=pltpu.VMEM)],
        out_specs=pl.BlockSpec(memory_space=pltpu.VMEM),
        scratch_shapes=[
            pltpu.VMEM((N_DEV, QR, 1024), jnp.bfloat16),
            pltpu.SemaphoreType.DMA((N_DEV - 1,)),
            pltpu.SemaphoreType.DMA((N_DEV,)),
            pltpu.SemaphoreType.DMA((N_DEV - 1,)),
            pltpu.SemaphoreType.DMA((N_DEV,)),
            pltpu.SemaphoreType.DMA(()),
        ],
        compiler_params=pltpu.CompilerParams(collective_id=1),
    )(part)


def _quant(t):
    s = jnp.max(jnp.abs(t), axis=-1, keepdims=True) / 127.0
    s = jnp.maximum(s, 1e-6).astype(jnp.bfloat16)
    q = jnp.clip(jnp.round(t / s.astype(jnp.float32)), -127, 127)
    return (
        q.astype(jnp.int8).reshape(NG_KV, NPH, BLK, 32 * DH),
        s[..., 0].reshape(NG_KV, NPH, BLK, N_DEV, HL).transpose(3, 0, 1, 2, 4),
    )


def kernel(x, Wq, K_ext, V_ext, Wo):
    x2 = x[0].astype(jnp.bfloat16)
    wq = Wq.astype(jnp.bfloat16)
    wo = Wo.astype(jnp.bfloat16)
    k8, ksc = _quant(K_ext[0])
    v8, vsc = _quant(V_ext[0])

    part = _fused_attn(x2, wq, wo, k8, ksc, v8, vsc)
    out = _allreduce(part)
    return out[None].astype(jnp.float32)


# device time: 318286 ns/iter; 1.5065x vs baseline; 1.5065x over previous
import jax
import jax.numpy as jnp
from jax import lax
from jax.experimental import pallas as pl
from jax.experimental.pallas import tpu as pltpu

N_DEV = 4
SQ = 2048
SKV_LOC = 2048
HL = 8
DH = 128
DLOC = HL * DH
NPH = 4
BLK = 64
NG_Q = SQ // (NPH * BLK)
NG_KV = SKV_LOC // (NPH * BLK)
QROWS = NG_Q * BLK
KROWS = N_DEV * NG_KV * BLK
SCALE = 0.08838834764831843


def _fused_body(x_ref, wq_ref, wo_ref, k_ref, ks_ref, v_ref, vs_ref, out_ref,
                kgbuf, vgbuf, ksbuf, vsbuf, send_sems, recv_sems, copy_sems):
    my = lax.axis_index("i")

    barrier = pltpu.get_barrier_semaphore()
    for di in range(1, N_DEV):
        pl.semaphore_signal(
            barrier, inc=1,
            device_id=((my + di) % N_DEV,),
            device_id_type=pl.DeviceIdType.MESH,
        )
    pl.semaphore_wait(barrier, N_DEV - 1)

    pairs = ((k_ref, kgbuf), (v_ref, vgbuf))
    spairs = ((ks_ref, ksbuf), (vs_ref, vsbuf))
    SS = (N_DEV - 1) * 2 * NPH
    RS_ = N_DEV * 2 * NPH

    copies = {}
    sends = []
    recvs = {}
    srecvs = {}
    for t, (src, dst) in enumerate(spairs):
        c = pltpu.make_async_copy(
            src.at[my],
            dst.at[my],
            copy_sems.at[2 * NPH + t],
        )
        c.start()
        copies[("s", t)] = c
        for di in range(1, N_DEV):
            peer = (my + di) % N_DEV
            r = pltpu.make_async_remote_copy(
                src_ref=src.at[peer],
                dst_ref=dst.at[my],
                send_sem=send_sems.at[SS + (di - 1) * 2 + t],
                recv_sem=recv_sems.at[RS_ + my * 2 + t],
                device_id=(peer,),
                device_id_type=pl.DeviceIdType.MESH,
            )
            r.start()
            sends.append(r)
            rr = pltpu.make_async_remote_copy(
                src_ref=src.at[peer],
                dst_ref=dst.at[peer],
                send_sem=send_sems.at[SS + t],
                recv_sem=recv_sems.at[RS_ + peer * 2 + t],
                device_id=(peer,),
                device_id_type=pl.DeviceIdType.MESH,
            )
            srecvs[(di, t)] = rr

    for p in range(NPH):
        for t, (src, dst) in enumerate(pairs):
            c = pltpu.make_async_copy(
                src.at[:, p, :, pl.ds(my * DLOC, DLOC)],
                dst.at[p, my],
                copy_sems.at[t * NPH + p],
            )
            c.start()
            copies[(t, p)] = c
        for di in range(1, N_DEV):
            peer = (my + di) % N_DEV
            for t, (src, dst) in enumerate(pairs):
                r = pltpu.make_async_remote_copy(
                    src_ref=src.at[:, p, :, pl.ds(peer * DLOC, DLOC)],
                    dst_ref=dst.at[p, my],
                    send_sem=send_sems.at[((di - 1) * 2 + t) * NPH + p],
                    recv_sem=recv_sems.at[(my * 2 + t) * NPH + p],
                    device_id=(peer,),
                    device_id_type=pl.DeviceIdType.MESH,
                )
                r.start()
                sends.append(r)
                src_dev = (my + di) % N_DEV
                rr = pltpu.make_async_remote_copy(
                    src_ref=src.at[:, p, :, pl.ds(src_dev * DLOC, DLOC)],
                    dst_ref=dst.at[p, src_dev],
                    send_sem=send_sems.at[t * NPH + p],
                    recv_sem=recv_sems.at[(src_dev * 2 + t) * NPH + p],
                    device_id=(src_dev,),
                    device_id_type=pl.DeviceIdType.MESH,
                )
                recvs[(di, t, p)] = rr

    q = jnp.dot(x_ref[:], wq_ref[:], preferred_element_type=jnp.float32)
    q = (q * SCALE).astype(jnp.bfloat16)

    for t in range(2):
        copies[("s", t)].wait()
        for di in range(1, N_DEV):
            srecvs[(di, t)].wait_recv()
    ksall = ksbuf[...]
    vsall = vsbuf[...]

    ctx_p = [None] * NPH
    for p in range(NPH):
        for t in range(2):
            copies[(t, p)].wait()
            for di in range(1, N_DEV):
                recvs[(di, t, p)].wait_recv()
        ks = ksall[:, :, p, :, :].reshape(KROWS, HL)
        vs = vsall[:, :, p, :, :].reshape(KROWS, HL)
        kp = kgbuf[p].reshape(KROWS, DLOC).astype(jnp.bfloat16)
        vp = vgbuf[p].reshape(KROWS, DLOC).astype(jnp.bfloat16)

        qp = jnp.concatenate(
            [q[g * NPH * BLK + p * BLK: g * NPH * BLK + (p + 1) * BLK, :]
             for g in range(NG_Q)],
            axis=0,
        )

        ctx_h = []
        for h in range(HL):
            qh = qp[:, h * DH:(h + 1) * DH]
            kh = kp[:, h * DH:(h + 1) * DH] * ks[:, h:h + 1]
            vh = vp[:, h * DH:(h + 1) * DH] * vs[:, h:h + 1]
            s = lax.dot_general(
                qh, kh, (((1,), (1,)), ((), ())),
                preferred_element_type=jnp.float32,
            )
            m = jnp.max(s, axis=1, keepdims=True)
            e = jnp.exp(s - m)
            l = jnp.sum(e, axis=1, keepdims=True)
            w = (e / l).astype(jnp.bfloat16)
            ctx = jnp.dot(w, vh, preferred_element_type=jnp.float32)
            ctx_h.append(ctx.astype(jnp.bfloat16))
        ctx_p[p] = jnp.concatenate(ctx_h, axis=1)

    rows = []
    for g in range(NG_Q):
        for p in range(NPH):
            rows.append(ctx_p[p][g * BLK:(g + 1) * BLK, :])
    ctx = jnp.concatenate(rows, axis=0)

    out = jnp.dot(ctx, wo_ref[:], preferred_element_type=jnp.float32)
    out_ref[:] = out.astype(jnp.bfloat16)

    for r in sends:
        r.wait_send()


def _fused_attn(x2, wq, wo, k8, ksc, v8, vsc):
    gshape = (NPH, N_DEV, NG_KV, BLK, DLOC)
    sshape = (N_DEV, NG_KV, NPH, BLK, HL)
    return pl.pallas_call(
        _fused_body,
        out_shape=jax.ShapeDtypeStruct((SQ, 1024), jnp.bfloat16),
        in_specs=[
            pl.BlockSpec(memory_space=pltpu.VMEM),
            pl.BlockSpec(memory_space=pltpu.VMEM),
            pl.BlockSpec(memory_space=pltpu.VMEM),
            pl.BlockSpec(memory_space=pl.ANY),
            pl.BlockSpec(memory_space=pl.ANY),
            pl.BlockSpec(memory_space=pl.ANY),
            pl.BlockSpec(memory_space=pl.ANY),
        ],
        out_specs=pl.BlockSpec(memory_space=pltpu.VMEM),
        scratch_shapes=[
            pltpu.VMEM(gshape, jnp.int8),
            pltpu.VMEM(gshape, jnp.int8),
            pltpu.VMEM(sshape, jnp.bfloat16),
            pltpu.VMEM(sshape, jnp.bfloat16),
            pltpu.SemaphoreType.DMA(((N_DEV - 1) * 2 * NPH
                                     + (N_DEV - 1) * 2,)),
            pltpu.SemaphoreType.DMA((N_DEV * 2 * NPH
                                     + N_DEV * 2,)),
            pltpu.SemaphoreType.DMA((2 * NPH + 2,)),
        ],
        compiler_params=pltpu.CompilerParams(
            collective_id=0, vmem_limit_bytes=64 * 1024 * 1024),
    )(x2, wq, wo, k8, ksc, v8, vsc)


QR = SQ // N_DEV


def _ar_body(part_ref, out_ref, rsbuf, rs_send, rs_recv,
             ag_send, ag_recv, csem):
    my = lax.axis_index("i")

    barrier = pltpu.get_barrier_semaphore()
    for di in range(1, N_DEV):
        pl.semaphore_signal(
            barrier, inc=1,
            device_id=((my + di) % N_DEV,),
            device_id_type=pl.DeviceIdType.MESH,
        )
    pl.semaphore_wait(barrier, N_DEV - 1)

    c = pltpu.make_async_copy(
        part_ref.at[pl.ds(my * QR, QR)], rsbuf.at[my], csem)
    c.start()
    rs_sends = []
    for di in range(1, N_DEV):
        peer = (my + di) % N_DEV
        r = pltpu.make_async_remote_copy(
            src_ref=part_ref.at[pl.ds(peer * QR, QR)],
            dst_ref=rsbuf.at[my],
            send_sem=rs_send.at[di - 1],
            recv_sem=rs_recv.at[my],
            device_id=(peer,),
            device_id_type=pl.DeviceIdType.MESH,
        )
        r.start()
        rs_sends.append(r)

    c.wait()
    for di in range(1, N_DEV):
        src_dev = (my + di) % N_DEV
        r = pltpu.make_async_remote_copy(
            src_ref=part_ref.at[pl.ds(0, QR)],
            dst_ref=rsbuf.at[src_dev],
            send_sem=rs_send.at[0],
            recv_sem=rs_recv.at[src_dev],
            device_id=(src_dev,),
            device_id_type=pl.DeviceIdType.MESH,
        )
        r.wait_recv()
    for r in rs_sends:
        r.wait_send()

    acc = rsbuf[...]
    red = (acc[0].astype(jnp.float32) + acc[1].astype(jnp.float32)
           + acc[2].astype(jnp.float32) + acc[3].astype(jnp.float32))
    out_ref[pl.ds(my * QR, QR), :] = red.astype(jnp.bfloat16)

    ag_sends = []
    for di in range(1, N_DEV):
        peer = (my + di) % N_DEV
        r = pltpu.make_async_remote_copy(
            src_ref=out_ref.at[pl.ds(my * QR, QR)],
            dst_ref=out_ref.at[pl.ds(my * QR, QR)],
            send_sem=ag_send.at[di - 1],
            recv_sem=ag_recv.at[my],
            device_id=(peer,),
            device_id_type=pl.DeviceIdType.MESH,
        )
        r.start()
        ag_sends.append(r)
    for di in range(1, N_DEV):
        src_dev = (my + di) % N_DEV
        r = pltpu.make_async_remote_copy(
            src_ref=out_ref.at[pl.ds(src_dev * QR, QR)],
            dst_ref=out_ref.at[pl.ds(src_dev * QR, QR)],
            send_sem=ag_send.at[0],
            recv_sem=ag_recv.at[src_dev],
            device_id=(src_dev,),
            device_id_type=pl.DeviceIdType.MESH,
        )
        r.wait_recv()
    for r in ag_sends:
        r.wait_send()


def _allreduce(part):
    return pl.pallas_call(
        _ar_body,
        out_shape=jax.ShapeDtypeStruct((SQ, 1024), jnp.bfloat16),
        in_specs=[pl.BlockSpec(memory_space=pltpu.VMEM)],
        out_specs=pl.BlockSpec(memory_space=pltpu.VMEM),
        scratch_shapes=[
            pltpu.VMEM((N_DEV, QR, 1024), jnp.bfloat16),
            pltpu.SemaphoreType.DMA((N_DEV - 1,)),
            pltpu.SemaphoreType.DMA((N_DEV,)),
            pltpu.SemaphoreType.DMA((N_DEV - 1,)),
            pltpu.SemaphoreType.DMA((N_DEV,)),
            pltpu.SemaphoreType.DMA(()),
        ],
        compiler_params=pltpu.CompilerParams(collective_id=1),
    )(part)


def _quant_body(t_ref, q_ref, s_ref, fbuf, obuf, sscr, isem, osem, ssem):
    cins = []
    for p in range(2):
        c = pltpu.make_async_copy(t_ref.at[:, p], fbuf.at[p % 2], isem.at[p % 2])
        c.start()
        cins.append(c)
    for p in range(NPH):
        cins[p].wait()
        v = fbuf[p % 2]
        if p + 2 < NPH:
            c = pltpu.make_async_copy(
                t_ref.at[:, p + 2], fbuf.at[p % 2], isem.at[p % 2])
            cins.append(c)
        s = jnp.max(jnp.abs(v), axis=-1) / 127.0
        s = jnp.maximum(s, 1e-6)
        q = jnp.clip(jnp.round(v / s[..., None]), -127, 127).astype(jnp.int8)
        obuf[...] = q
        sb = s.astype(jnp.bfloat16)
        for g in range(N_DEV):
            sscr[g] = sb[:, :, g * HL:(g + 1) * HL]
        cout = pltpu.make_async_copy(obuf, q_ref.at[:, p], osem)
        cout.start()
        scs = []
        for g in range(N_DEV):
            cs = pltpu.make_async_copy(
                sscr.at[g], s_ref.at[g, :, p], ssem.at[g])
            cs.start()
            scs.append(cs)
        cout.wait()
        for cs in scs:
            cs.wait()
        if p + 2 < NPH:
            cins[p + 2].start()


def _quant_pl(t):
    return pl.pallas_call(
        _quant_body,
        out_shape=(
            jax.ShapeDtypeStruct((NG_KV, NPH, BLK, 32, DH), jnp.int8),
            jax.ShapeDtypeStruct((N_DEV, NG_KV, NPH, BLK, HL), jnp.bfloat16),
        ),
        in_specs=[pl.BlockSpec(memory_space=pl.ANY)],
        out_specs=(
            pl.BlockSpec(memory_space=pl.ANY),
            pl.BlockSpec(memory_space=pl.ANY),
        ),
        scratch_shapes=[
            pltpu.VMEM((2, NG_KV, BLK, 32, DH), jnp.float32),
            pltpu.VMEM((NG_KV, BLK, 32, DH), jnp.int8),
            pltpu.VMEM((N_DEV, NG_KV, BLK, HL), jnp.bfloat16),
            pltpu.SemaphoreType.DMA((2,)),
            pltpu.SemaphoreType.DMA(()),
            pltpu.SemaphoreType.DMA((N_DEV,)),
        ],
    )(t)


def kernel(x, Wq, K_ext, V_ext, Wo):
    x2 = x[0].astype(jnp.bfloat16)
    wq = Wq.astype(jnp.bfloat16)
    wo = Wo.astype(jnp.bfloat16)
    kt = K_ext[0].reshape(NG_KV, NPH, BLK, 32, DH)
    vt = V_ext[0].reshape(NG_KV, NPH, BLK, 32, DH)
    k8, ksc = _quant_pl(kt)
    v8, vsc = _quant_pl(vt)
    k8 = k8.reshape(NG_KV, NPH, BLK, 32 * DH)
    v8 = v8.reshape(NG_KV, NPH, BLK, 32 * DH)

    part = _fused_attn(x2, wq, wo, k8, ksc, v8, vsc)
    out = _allreduce(part)
    return out[None].astype(jnp.float32)
